# baseline (speedup 1.0000x reference)
import functools

import jax
import jax.numpy as jnp
from jax import lax
from jax.experimental import pallas as pl
from jax.experimental.pallas import tpu as pltpu
from jax.experimental.pallas import tpu_sc as plsc

D_MODEL = 512
MAX_LEN = 12
BATCH = 4096
NUM_CORES = 2
NUM_SUBCORES = 16
NUM_WORKERS = NUM_CORES * NUM_SUBCORES
LENS_PER_WORKER = BATCH // NUM_WORKERS
ROWS_PER_WORKER = LENS_PER_WORKER * MAX_LEN
CHUNK_ELEMS = 2
CHUNK_ROWS = CHUNK_ELEMS * MAX_LEN
CHUNK_WORDS = CHUNK_ROWS * D_MODEL
NUM_CHUNKS = ROWS_PER_WORKER // CHUNK_ROWS
LANES = 16
LEN_GROUPS = LENS_PER_WORKER // LANES
VREGS_PER_ROW = D_MODEL // LANES


def _pe_body(lens_hbm, table_hbm, out_hbm, lens_v, mask_v, pe_v, stag2, sem0):
    wid = lax.axis_index("s") * NUM_CORES + lax.axis_index("c")
    base_len = wid * LENS_PER_WORKER
    pltpu.sync_copy(lens_hbm.at[pl.ds(base_len, LENS_PER_WORKER)], lens_v)
    pltpu.sync_copy(table_hbm.at[pl.ds(0, LANES)], pe_v)

    lane = lax.iota(jnp.int32, LANES)
    ge = [lane >= (MAX_LEN - 4 * v) for v in range(3)]
    j_vec = [lane + (4 * v) - jnp.where(ge[v], MAX_LEN, 0) for v in range(3)]
    bstep = [jnp.where(ge[v], v + 1, v) for v in range(3)]
    ones = jnp.full((LANES,), 1.0, jnp.float32)
    zeros = jnp.zeros((LANES,), jnp.float32)
    for g in range(LEN_GROUPS):
        grp = lens_v[pl.ds(g * LANES, LANES)]
        for w in range(MAX_LEN):
            v = w % 3
            b_local = bstep[v] + 4 * (w // 3)
            len_vec = lax.gather(
                grp,
                b_local[:, None],
                lax.GatherDimensionNumbers(
                    offset_dims=(), collapsed_slice_dims=(0,), start_index_map=(0,)
                ),
                (1,),
                mode=lax.GatherScatterMode.PROMISE_IN_BOUNDS,
            )
            mask_vec = jnp.where(j_vec[v] < len_vec, ones, zeros)
            mask_v[pl.ds(g * 192 + w * LANES, LANES)] = mask_vec

    elem_base = wid * LENS_PER_WORKER

    gd = lax.GatherDimensionNumbers(
        offset_dims=(), collapsed_slice_dims=(0,), start_index_map=(0,)
    )
    splats = [lane * 0 + r for r in range(LANES)]

    def _drain_one():
        pltpu.make_async_copy(
            out_hbm.at[pl.ds(0, CHUNK_ELEMS)], stag2.at[pl.ds(0, CHUNK_ELEMS)], sem0
        ).wait()

    def chunk_body(c, carry):
        @pl.when(c >= 2)
        def _():
            _drain_one()

        def do_half(half_elems):
            mg0 = mask_v[pl.ds(c * CHUNK_ROWS, LANES)]
            mg1 = mask_v[pl.ds(c * CHUNK_ROWS + LANES, LANES)]
            for q in range(CHUNK_ROWS):
                grp, idx = (mg0, q) if q < LANES else (mg1, q - LANES)
                m = lax.gather(
                    grp,
                    splats[idx][:, None],
                    gd,
                    (1,),
                    mode=lax.GatherScatterMode.PROMISE_IN_BOUNDS,
                )
                e, j = divmod(q, MAX_LEN)
                for k in range(VREGS_PER_ROW):
                    stag2[half_elems + e, j, pl.ds(k * LANES, LANES)] = (
                        pe_v[j + 1, pl.ds(k * LANES, LANES)] * m
                    )
            pltpu.async_copy(
                stag2.at[pl.ds(half_elems, CHUNK_ELEMS)],
                out_hbm.at[pl.ds(elem_base + c * CHUNK_ELEMS, CHUNK_ELEMS)],
                sem0,
            )

        @pl.when((c & 1) == 0)
        def _():
            do_half(0)

        @pl.when((c & 1) == 1)
        def _():
            do_half(CHUNK_ELEMS)

        return carry

    lax.fori_loop(0, NUM_CHUNKS, chunk_body, 0)
    _drain_one()
    _drain_one()


_pe_call = functools.partial(
    pl.kernel,
    mesh=plsc.VectorSubcoreMesh(core_axis_name="c", subcore_axis_name="s"),
    out_type=jax.ShapeDtypeStruct((BATCH, MAX_LEN, D_MODEL), jnp.float32),
    scratch_types=[
        pltpu.VMEM((LENS_PER_WORKER,), jnp.int32),
        pltpu.VMEM((ROWS_PER_WORKER + 2 * LANES,), jnp.float32),
        pltpu.VMEM((LANES, D_MODEL), jnp.float32),
        pltpu.VMEM((2 * CHUNK_ELEMS, MAX_LEN, D_MODEL), jnp.float32),
        pltpu.SemaphoreType.DMA,
    ],
    compiler_params=pltpu.CompilerParams(
        use_tc_tiling_on_sc=True,
        skip_device_barrier=True,
        disable_bounds_checks=True,
        disable_semaphore_checks=True,
    ),
)(_pe_body)


def kernel(input_lens, table):
    return _pe_call(input_lens, table)


# speedup vs baseline: 1.0033x; 1.0033x over previous
import functools

import jax
import jax.numpy as jnp
from jax import lax
from jax.experimental import pallas as pl
from jax.experimental.pallas import tpu as pltpu
from jax.experimental.pallas import tpu_sc as plsc

D_MODEL = 512
MAX_LEN = 12
BATCH = 4096
NUM_CORES = 2
NUM_SUBCORES = 16
NUM_WORKERS = NUM_CORES * NUM_SUBCORES
LENS_PER_WORKER = BATCH // NUM_WORKERS
ROWS_PER_WORKER = LENS_PER_WORKER * MAX_LEN
CHUNK_ELEMS = 2
CHUNK_ROWS = CHUNK_ELEMS * MAX_LEN
CHUNK_WORDS = CHUNK_ROWS * D_MODEL
NUM_CHUNKS = ROWS_PER_WORKER // CHUNK_ROWS
LANES = 16
LEN_GROUPS = LENS_PER_WORKER // LANES
VREGS_PER_ROW = D_MODEL // LANES


def _pe_body(lens_hbm, table_hbm, out_hbm, lens_v, mask_v, pe_v, stag2, sem0):
    wid = lax.axis_index("s") * NUM_CORES + lax.axis_index("c")
    base_len = wid * LENS_PER_WORKER
    pltpu.sync_copy(lens_hbm.at[pl.ds(base_len, LENS_PER_WORKER)], lens_v)
    pltpu.sync_copy(table_hbm.at[pl.ds(0, LANES)], pe_v)

    lane = lax.iota(jnp.int32, LANES)
    ge = [lane >= (MAX_LEN - 4 * v) for v in range(3)]
    j_vec = [lane + (4 * v) - jnp.where(ge[v], MAX_LEN, 0) for v in range(3)]
    bstep = [jnp.where(ge[v], v + 1, v) for v in range(3)]
    ones = jnp.full((LANES,), 1.0, jnp.float32)
    zeros = jnp.zeros((LANES,), jnp.float32)
    for g in range(LEN_GROUPS):
        grp = lens_v[pl.ds(g * LANES, LANES)]
        for w in range(MAX_LEN):
            v = w % 3
            b_local = bstep[v] + 4 * (w // 3)
            len_vec = lax.gather(
                grp,
                b_local[:, None],
                lax.GatherDimensionNumbers(
                    offset_dims=(), collapsed_slice_dims=(0,), start_index_map=(0,)
                ),
                (1,),
                mode=lax.GatherScatterMode.PROMISE_IN_BOUNDS,
            )
            mask_vec = jnp.where(j_vec[v] < len_vec, ones, zeros)
            mask_v[pl.ds(g * 192 + w * LANES, LANES)] = mask_vec

    elem_base = wid * LENS_PER_WORKER

    gd = lax.GatherDimensionNumbers(
        offset_dims=(), collapsed_slice_dims=(0,), start_index_map=(0,)
    )
    splats = [lane * 0 + r for r in range(LANES)]

    def _drain_one():
        pltpu.make_async_copy(
            out_hbm.at[pl.ds(0, CHUNK_ELEMS)], stag2.at[pl.ds(0, CHUNK_ELEMS)], sem0
        ).wait()

    def chunk_body(c, carry):
        @pl.when(c >= 2)
        def _():
            _drain_one()

        def do_half(half_elems):
            mg0 = mask_v[pl.ds(c * CHUNK_ROWS, LANES)]
            mg1 = mask_v[pl.ds(c * CHUNK_ROWS + LANES, LANES)]
            for q in range(CHUNK_ROWS):
                grp, idx = (mg0, q) if q < LANES else (mg1, q - LANES)
                m = lax.gather(
                    grp,
                    splats[idx][:, None],
                    gd,
                    (1,),
                    mode=lax.GatherScatterMode.PROMISE_IN_BOUNDS,
                )
                e, j = divmod(q, MAX_LEN)
                for k in range(VREGS_PER_ROW):
                    stag2[half_elems + e, j, pl.ds(k * LANES, LANES)] = (
                        pe_v[j + 1, pl.ds(k * LANES, LANES)] * m
                    )
            pltpu.async_copy(
                stag2.at[pl.ds(half_elems, CHUNK_ELEMS)],
                out_hbm.at[pl.ds(elem_base + c * CHUNK_ELEMS, CHUNK_ELEMS)],
                sem0,
            )

        @pl.when((c & 1) == 0)
        def _():
            do_half(0)

        @pl.when((c & 1) == 1)
        def _():
            do_half(CHUNK_ELEMS)

        return carry

    lax.fori_loop(0, NUM_CHUNKS, chunk_body, 0)
    _drain_one()
    _drain_one()


_pe_call = functools.partial(
    pl.kernel,
    mesh=plsc.VectorSubcoreMesh(core_axis_name="c", subcore_axis_name="s"),
    out_type=jax.ShapeDtypeStruct((BATCH, MAX_LEN, D_MODEL), jnp.float32),
    scratch_types=[
        pltpu.VMEM((LENS_PER_WORKER,), jnp.int32),
        pltpu.VMEM((ROWS_PER_WORKER + 2 * LANES,), jnp.float32),
        pltpu.VMEM((LANES, D_MODEL), jnp.float32),
        pltpu.VMEM((2 * CHUNK_ELEMS, MAX_LEN, D_MODEL), jnp.float32),
        pltpu.SemaphoreType.DMA,
    ],
    compiler_params=pltpu.CompilerParams(use_tc_tiling_on_sc=True),
)(_pe_body)


def kernel(input_lens, table):
    return _pe_call(input_lens, table)
